# baseline (speedup 1.0000x reference)
import functools

import jax
import jax.numpy as jnp
from jax import lax
from jax.experimental import pallas as pl
from jax.experimental.pallas import tpu as pltpu
from jax.experimental.pallas import tpu_sc as plsc

NTOKEN = 100000
NINP = 128
NREL = 64
N = 4096
L = 200

NC = 2
NS = 16
NW = NC * NS
SEQ_PER_W = N // NW
GA = 96
GB = L - GA
BCH = NREL // 32
UNROLL = 4

PROJ_BLK = 1000
HALF = NTOKEN // 2


def _project_body(emb_lo, emb_hi, w_ref, out_ref):
    a = lax.dot_general(emb_lo[...], w_ref[...], (((1,), (1,)), ((), ())),
                        preferred_element_type=jnp.float32)
    b = lax.dot_general(emb_hi[...], w_ref[...], (((1,), (1,)), ((), ())),
                        preferred_element_type=jnp.float32)
    out_ref[...] = jnp.concatenate([a, b], axis=1).astype(jnp.bfloat16)


def _project_fn(emb_table, w):
    grid = HALF // PROJ_BLK
    return pl.pallas_call(
        _project_body,
        grid=(grid,),
        in_specs=[pl.BlockSpec((PROJ_BLK, NINP), lambda i: (i, 0)),
                  pl.BlockSpec((PROJ_BLK, NINP), lambda i: (i + grid, 0)),
                  pl.BlockSpec((NREL, NINP), lambda i: (0, 0))],
        out_specs=pl.BlockSpec((PROJ_BLK, 2 * NREL), lambda i: (i, 0)),
        out_shape=jax.ShapeDtypeStruct((HALF, 2 * NREL), jnp.bfloat16),
    )(emb_table, emb_table, w)


def _gather_pool_body(seq_hbm, table_hbm, out_hbm, idx_all, rows, out_v,
                      sem0, sem1):
    wid = lax.axis_index("s") * NC + lax.axis_index("c")
    base = wid * SEQ_PER_W
    sems = (sem0, sem1)

    pltpu.sync_copy(seq_hbm.at[pl.ds(base * L, SEQ_PER_W * L)], idx_all)

    def mk(i, buf):
        return (pltpu.make_async_copy(
                    table_hbm.at[idx_all.at[pl.ds(i * L, GA)]],
                    rows.at[buf, pl.ds(0, GA)], sems[buf]),
                pltpu.make_async_copy(
                    table_hbm.at[idx_all.at[pl.ds(i * L + GA, GB)]],
                    rows.at[buf, pl.ds(GA, GB)], sems[buf]))

    def fire(i, buf):
        a, c = mk(i, buf)
        a.start()
        c.start()

    def drain(i, buf):
        a, c = mk(i, buf)
        a.wait()
        c.wait()

    def accumulate(buf, i):
        def acc_body(r, acc):
            out = []
            for d in range(BCH):
                for u in range(UNROLL):
                    v = acc[d * UNROLL + u]
                    v = v + rows[buf, UNROLL * r + u, pl.ds(d * 32, 32)]
                    out.append(v)
            return tuple(out)

        zero = jnp.zeros((32,), jnp.bfloat16)
        acc = lax.fori_loop(0, L // UNROLL, acc_body, (zero,) * (BCH * UNROLL))
        for d in range(BCH):
            tot = acc[d * UNROLL]
            for u in range(1, UNROLL):
                tot = tot + acc[d * UNROLL + u]
            out_v[pl.ds(i * NREL + d * 32, 32)] = tot

    fire(0, 0)

    def g_body(g, carry):
        fire(2 * g + 1, 1)
        drain(2 * g, 0)
        accumulate(0, 2 * g)

        @pl.when(g < SEQ_PER_W // 2 - 1)
        def _():
            fire(2 * g + 2, 0)

        drain(2 * g + 1, 1)
        accumulate(1, 2 * g + 1)
        return carry

    lax.fori_loop(0, SEQ_PER_W // 2, g_body, 0)
    pltpu.sync_copy(out_v, out_hbm.at[pl.ds(base * NREL, SEQ_PER_W * NREL)])


@functools.lru_cache(maxsize=None)
def _gather_pool_fn():
    mesh = plsc.VectorSubcoreMesh(core_axis_name="c", subcore_axis_name="s",
                                  num_cores=NC, num_subcores=NS)
    return pl.kernel(
        _gather_pool_body,
        out_type=jax.ShapeDtypeStruct((N * NREL,), jnp.bfloat16),
        mesh=mesh,
        compiler_params=pltpu.CompilerParams(use_tc_tiling_on_sc=False),
        scratch_types=[
            pltpu.VMEM((SEQ_PER_W * L,), jnp.int32),
            pltpu.VMEM((2, L, NREL), jnp.bfloat16),
            pltpu.VMEM((SEQ_PER_W * NREL,), jnp.bfloat16),
            pltpu.SemaphoreType.DMA,
            pltpu.SemaphoreType.DMA,
        ],
    )


def _stats_body(sums_ref, b_ref, rel_ref, logp_ref, acc_ref):
    logit = sums_ref[...].astype(jnp.float32) + b_ref[...]
    cols = lax.broadcasted_iota(jnp.int32, (N, NREL), 1)
    mask = cols == rel_ref[...]
    t = jnp.exp(-jnp.abs(logit))
    log_sig = jnp.minimum(logit, 0.0) - jnp.log1p(t)
    sig = jnp.where(logit >= 0, 1.0 / (1.0 + t), t / (1.0 + t))
    other = jnp.log(1.0 + 1e-05 - sig)
    logp_ref[0, 0] = jnp.sum(jnp.where(mask, log_sig, other)) / N
    agree = ((logit > 0.5) == mask).astype(jnp.float32)
    acc_ref[0, 0] = jnp.sum(agree) / (N * NREL)


def _stats_fn(sums, b2, rel2):
    return pl.pallas_call(
        _stats_body,
        out_shape=(jax.ShapeDtypeStruct((1, 1), jnp.float32),
                   jax.ShapeDtypeStruct((1, 1), jnp.float32)),
        out_specs=(pl.BlockSpec(memory_space=pltpu.SMEM),
                   pl.BlockSpec(memory_space=pltpu.SMEM)),
    )(sums, b2, rel2)


def kernel(seq, masks, n, tok, n_idx, idx, m, src, dst, rel, emb_table, W, b):
    proj = _project_fn(emb_table, W).reshape(NTOKEN, NREL)
    s = seq.astype(jnp.int32)
    s2 = s * 2 - jnp.where(s >= HALF, NTOKEN - 1, 0)
    seq_flat = s2.reshape(N * L)
    sums = _gather_pool_fn()(seq_flat, proj).reshape(N, NREL)
    logp, acc = _stats_fn(sums, b.reshape(1, NREL),
                          rel.astype(jnp.int32).reshape(N, 1))
    return logp[0, 0], acc[0, 0]


# speedup vs baseline: 1.1565x; 1.1565x over previous
import functools

import jax
import jax.numpy as jnp
from jax import lax
from jax.experimental import pallas as pl
from jax.experimental.pallas import tpu as pltpu
from jax.experimental.pallas import tpu_sc as plsc

NTOKEN = 100000
NINP = 128
NREL = 64
N = 4096
L = 200

NC = 2
NS = 16
NW = NC * NS
SEQ_PER_W = N // NW
GS = 4
NG = SEQ_PER_W // GS
GROUP_ROWS = GS * L
BCH = NREL // 32
UNROLL = 4

PROJ_BLK = 1000
HALF = NTOKEN // 2


def _project_body(emb_lo, emb_hi, w_ref, out_ref):
    a = lax.dot_general(emb_lo[...], w_ref[...], (((1,), (1,)), ((), ())),
                        preferred_element_type=jnp.float32)
    b = lax.dot_general(emb_hi[...], w_ref[...], (((1,), (1,)), ((), ())),
                        preferred_element_type=jnp.float32)
    out_ref[...] = jnp.concatenate([a, b], axis=1).astype(jnp.bfloat16)


def _project_fn(emb_table, w):
    grid = HALF // PROJ_BLK
    return pl.pallas_call(
        _project_body,
        grid=(grid,),
        in_specs=[pl.BlockSpec((PROJ_BLK, NINP), lambda i: (i, 0)),
                  pl.BlockSpec((PROJ_BLK, NINP), lambda i: (i + grid, 0)),
                  pl.BlockSpec((NREL, NINP), lambda i: (0, 0))],
        out_specs=pl.BlockSpec((PROJ_BLK, 2 * NREL), lambda i: (i, 0)),
        out_shape=jax.ShapeDtypeStruct((HALF, 2 * NREL), jnp.bfloat16),
    )(emb_table, emb_table, w)


def _gather_pool_body(seq_hbm, table_hbm, out_hbm, idx_all, rows, out_v,
                      sem0, sem1):
    wid = lax.axis_index("s") * NC + lax.axis_index("c")
    base = wid * SEQ_PER_W
    sems = (sem0, sem1)

    pltpu.sync_copy(seq_hbm.at[pl.ds(wid * NG, NG)], idx_all)

    def mk(g, buf):
        return pltpu.make_async_copy(
            table_hbm.at[idx_all.at[g]], rows.at[buf], sems[buf])

    def fire(g, buf):
        mk(g, buf).start()

    def drain(g, buf):
        mk(g, buf).wait()

    zero = jnp.zeros((32,), jnp.bfloat16)

    def accumulate(buf, g):
        for j in range(GS):
            def acc_body(r, a, _j=j):
                out = []
                for d in range(BCH):
                    for u in range(UNROLL):
                        v = a[d * UNROLL + u]
                        v = v + rows[buf, _j * L + UNROLL * r + u,
                                     pl.ds(d * 32, 32)]
                        out.append(v)
                return tuple(out)

            acc = lax.fori_loop(0, L // UNROLL, acc_body,
                                (zero,) * (BCH * UNROLL))
            for d in range(BCH):
                tot = acc[d * UNROLL]
                for u in range(1, UNROLL):
                    tot = tot + acc[d * UNROLL + u]
                out_v[pl.ds((g * GS + j) * NREL + d * 32, 32)] = tot

    fire(0, 0)

    def g_body(g, carry):
        fire(2 * g + 1, 1)
        drain(2 * g, 0)
        accumulate(0, 2 * g)

        @pl.when(g < NG // 2 - 1)
        def _():
            fire(2 * g + 2, 0)

        drain(2 * g + 1, 1)
        accumulate(1, 2 * g + 1)
        return carry

    lax.fori_loop(0, NG // 2, g_body, 0)
    pltpu.sync_copy(out_v, out_hbm.at[pl.ds(base * NREL, SEQ_PER_W * NREL)])


@functools.lru_cache(maxsize=None)
def _gather_pool_fn():
    mesh = plsc.VectorSubcoreMesh(core_axis_name="c", subcore_axis_name="s",
                                  num_cores=NC, num_subcores=NS)
    return pl.kernel(
        _gather_pool_body,
        out_type=jax.ShapeDtypeStruct((N * NREL,), jnp.bfloat16),
        mesh=mesh,
        compiler_params=pltpu.CompilerParams(use_tc_tiling_on_sc=False),
        scratch_types=[
            pltpu.VMEM((NG, GROUP_ROWS), jnp.int32),
            pltpu.VMEM((2, GROUP_ROWS, NREL), jnp.bfloat16),
            pltpu.VMEM((SEQ_PER_W * NREL,), jnp.bfloat16),
            pltpu.SemaphoreType.DMA,
            pltpu.SemaphoreType.DMA,
        ],
    )


def _stats_body(sums_ref, b_ref, rel_ref, logp_ref, acc_ref):
    logit = sums_ref[...].astype(jnp.float32) + b_ref[...]
    cols = lax.broadcasted_iota(jnp.int32, (N, NREL), 1)
    mask = cols == rel_ref[...]
    t = jnp.exp(-jnp.abs(logit))
    log_sig = jnp.minimum(logit, 0.0) - jnp.log1p(t)
    sig = jnp.where(logit >= 0, 1.0 / (1.0 + t), t / (1.0 + t))
    other = jnp.log(1.0 + 1e-05 - sig)
    logp_ref[0, 0] = jnp.sum(jnp.where(mask, log_sig, other)) / N
    agree = ((logit > 0.5) == mask).astype(jnp.float32)
    acc_ref[0, 0] = jnp.sum(agree) / (N * NREL)


def _stats_fn(sums, b2, rel2):
    return pl.pallas_call(
        _stats_body,
        out_shape=(jax.ShapeDtypeStruct((1, 1), jnp.float32),
                   jax.ShapeDtypeStruct((1, 1), jnp.float32)),
        out_specs=(pl.BlockSpec(memory_space=pltpu.SMEM),
                   pl.BlockSpec(memory_space=pltpu.SMEM)),
    )(sums, b2, rel2)


def kernel(seq, masks, n, tok, n_idx, idx, m, src, dst, rel, emb_table, W, b):
    proj = _project_fn(emb_table, W).reshape(NTOKEN, NREL)
    s = seq.astype(jnp.int32)
    s2 = s * 2 - jnp.where(s >= HALF, NTOKEN - 1, 0)
    seq_grp = s2.reshape(N // GS, GROUP_ROWS)
    sums = _gather_pool_fn()(seq_grp, proj).reshape(N, NREL)
    logp, acc = _stats_fn(sums, b.reshape(1, NREL),
                          rel.astype(jnp.int32).reshape(N, 1))
    return logp[0, 0], acc[0, 0]
